# baseline (speedup 1.0000x reference)
import functools

import jax
import jax.numpy as jnp
from jax import lax
from jax.experimental import pallas as pl
from jax.experimental.pallas import tpu as pltpu
from jax.experimental.pallas import tpu_sc as plsc

EMBED_DIM = 64
BATCH = 16384
CHUNK = 32

_info = plsc.get_sparse_core_info()
_NC, _NS = _info.num_cores, _info.num_subcores
_NW = _NC * _NS


def _make_gather(dim, batch, pairs_half):
    b_per_w = batch // _NW
    n_chunks = b_per_w // CHUNK
    pair_w = 2 * dim
    mesh = plsc.VectorSubcoreMesh(core_axis_name="c", subcore_axis_name="s")

    @functools.partial(
        pl.kernel,
        mesh=mesh,
        out_type=jax.ShapeDtypeStruct((batch, dim), jnp.float32),
        scratch_types=[
            pltpu.VMEM((b_per_w,), jnp.int32),
            pltpu.VMEM((b_per_w,), jnp.int32),
            pltpu.VMEM((b_per_w,), jnp.int32),
            pltpu.VMEM((b_per_w,), jnp.int32),
            pltpu.VMEM((2 * CHUNK, pair_w), jnp.float32),
            pltpu.VMEM((2 * CHUNK, pair_w), jnp.float32),
            pltpu.VMEM((CHUNK, dim), jnp.float32),
            pltpu.VMEM((CHUNK, dim), jnp.float32),
            pltpu.SemaphoreType.DMA,
            pltpu.SemaphoreType.DMA,
        ],
    )
    def gather_kernel(idx_hbm, ta_hbm, tb_hbm, out_hbm, pa_v, pb_v, off_v,
                      radd_v, tiles0, tiles1, rows0, rows1, sem0, sem1):
        wid = lax.axis_index("s") * _NC + lax.axis_index("c")
        base = wid * b_per_w
        pltpu.sync_copy(idx_hbm.at[pl.ds(base, b_per_w)], pa_v)

        def index_math(g, carry):
            v = pa_v[pl.ds(g * 16, 16)]
            p = lax.shift_right_logical(v, 1)
            h = lax.bitwise_and(v, 1)
            in_b = p >= pairs_half
            pb_v[pl.ds(g * 16, 16)] = jnp.where(in_b, p - pairs_half, 0)
            off_v[pl.ds(g * 16, 16)] = h * dim
            radd_v[pl.ds(g * 16, 16)] = jnp.where(in_b, CHUNK, 0)
            return carry

        lax.fori_loop(0, b_per_w // 16, index_math, 0)

        def index_math2(g, carry):
            v = pa_v[pl.ds(g * 16, 16)]
            p = lax.shift_right_logical(v, 1)
            pa_v[pl.ds(g * 16, 16)] = jnp.where(p < pairs_half, p, 0)
            return carry

        lax.fori_loop(0, b_per_w // 16, index_math2, 0)

        def fire(c, tiles, sem):
            da = pltpu.async_copy(
                ta_hbm.at[pa_v.at[pl.ds(c * CHUNK, CHUNK)]],
                tiles.at[pl.ds(0, CHUNK)], sem)
            db = pltpu.async_copy(
                tb_hbm.at[pb_v.at[pl.ds(c * CHUNK, CHUNK)]],
                tiles.at[pl.ds(CHUNK, CHUNK)], sem)
            return da, db

        def select(c, tiles, rows):
            for g2 in range(CHUNK // 16):
                ovec = off_v[pl.ds(c * CHUNK + g2 * 16, 16)]
                rvec = radd_v[pl.ds(c * CHUNK + g2 * 16, 16)]
                for l in range(16):
                    j = g2 * 16 + l
                    off = ovec[l]
                    row = rvec[l] + j
                    for kk in range(dim // 16):
                        rows[j, pl.ds(kk * 16, 16)] = (
                            tiles[row, pl.ds(off + kk * 16, 16)]
                        )
            pltpu.sync_copy(
                rows, out_hbm.at[pl.ds(base + c * CHUNK, CHUNK)]
            )

        def pair_loop(p, carry):
            c0 = p * 2
            c1 = c0 + 1
            da0, db0 = fire(c0, tiles0, sem0)
            da1, db1 = fire(c1, tiles1, sem1)
            da0.wait()
            db0.wait()
            select(c0, tiles0, rows0)
            da1.wait()
            db1.wait()
            select(c1, tiles1, rows1)
            return carry

        lax.fori_loop(0, n_chunks // 2, pair_loop, 0)

    return gather_kernel


@jax.jit
def kernel(user_inputs, table):
    num_rows, dim = table.shape
    half = num_rows // 2
    ta = table[:half].reshape(half // 2, 2 * dim)
    tb = table[half:].reshape(half // 2, 2 * dim)
    gather = _make_gather(dim, user_inputs.shape[0], half // 2)
    return gather(user_inputs.astype(jnp.int32), ta, tb)


# speedup vs baseline: 1.2602x; 1.2602x over previous
import functools

import jax
import jax.numpy as jnp
from jax import lax
from jax.experimental import pallas as pl
from jax.experimental.pallas import tpu as pltpu
from jax.experimental.pallas import tpu_sc as plsc

EMBED_DIM = 64
BATCH = 16384

_info = plsc.get_sparse_core_info()
_NC, _NS = _info.num_cores, _info.num_subcores

_BLK = 512
_R_MAIN = 62464
_FLUSH = 64
_HPAD = BATCH + 64
_NBINS = 128


def _make_sweep(num_rows, dim, batch):
    chalf = dim // _NC
    mesh = plsc.VectorSubcoreMesh(core_axis_name="c", subcore_axis_name="s")

    @functools.partial(
        pl.kernel,
        mesh=mesh,
        out_type=jax.ShapeDtypeStruct((_NC * _HPAD, 128), jnp.float32),
        scratch_types=[
            pltpu.VMEM((chalf, _BLK), jnp.float32),
            pltpu.VMEM((chalf, _BLK), jnp.float32),
            pltpu.VMEM((2048,), jnp.int32),
            pltpu.VMEM((batch + 16,), jnp.int32),
            pltpu.VMEM((batch + 16,), jnp.int32),
            pltpu.VMEM((batch + 16,), jnp.int32),
            pltpu.VMEM((batch + 16,), jnp.int32),
            pltpu.VMEM((_NBINS + 16,), jnp.int32),
            pltpu.VMEM((_NBINS + 16,), jnp.int32),
            pltpu.VMEM((_NBINS + 16,), jnp.int32),
            pltpu.VMEM((16,), jnp.int32),
            pltpu.VMEM((_FLUSH, 128), jnp.float32),
            pltpu.VMEM((_FLUSH,), jnp.int32),
            pltpu.SemaphoreType.DMA,
            pltpu.SemaphoreType.DMA,
            pltpu.SemaphoreType.DMA,
        ],
        compiler_params=pltpu.CompilerParams(needs_layout_passes=False),
    )
    def sweep_kernel(idx_hbm, tt_hbm, tab_hbm, out_hbm, slab_a, slab_b,
                     idx_c, loc_r, loc_b, srt_r, srt_b, hist, bounds,
                     cursor, bins_v, rows_p, bpad, sem_a, sem_b, sem_s):
        core = lax.axis_index("c")
        sub = lax.axis_index("s")
        c0 = core * chalf
        tile_r0 = pl.multiple_of(sub * _R_MAIN, 128)
        is_last = sub == _NS - 1
        r_range = jnp.where(is_last, num_rows - (_NS - 1) * _R_MAIN,
                            _R_MAIN)
        hbase = core * _HPAD
        dump = hbase + batch + sub
        lane = lax.iota(jnp.int32, 16)
        pad_slot = jnp.int32(batch + 8)
        dump_bin = jnp.int32(_NBINS - 1)
        tail_bin = 123

        def fire(dma_r0, slab, sem):
            pltpu.async_copy(
                tt_hbm.at[pl.ds(c0, chalf),
                          pl.ds(pl.multiple_of(tile_r0 + dma_r0, 128),
                                _BLK)],
                slab, sem)

        def drain(slab, sem):
            pltpu.make_async_copy(
                tt_hbm.at[pl.ds(c0, chalf), pl.ds(tile_r0, _BLK)],
                slab, sem).wait()

        fire(0, slab_a, sem_a)

        def scan_chunk(ch, cnt):
            pltpu.sync_copy(idx_hbm.at[pl.ds(ch * 2048, 2048)], idx_c)

            def scan_vec(i, cnt):
                v = idx_c[pl.ds(i * 16, 16)]
                vr = v - tile_r0
                m = (vr >= 0) & (vr < r_range)
                b = ch * 2048 + i * 16 + lane
                cs = plsc.cumsum(m.astype(jnp.int32))
                pos = jnp.where(m, cnt + cs - 1, pad_slot)
                plsc.store_scatter(loc_r, [pos], vr)
                plsc.store_scatter(loc_b, [pos], b)
                return cnt + cs[15]

            return lax.fori_loop(0, 2048 // 16, scan_vec, cnt)

        cnt = lax.fori_loop(0, batch // 2048, scan_chunk, jnp.int32(0))
        nvec = (cnt + 15) // 16

        rot_b = [lax.rem(lane - s + 16, 16) for s in range(1, 16)]
        rot_f = [lax.rem(lane + s, 16) for s in range(1, 16)]

        def dup_rank(bins):
            bins_v[pl.ds(0, 16)] = bins
            dup = jnp.zeros((16,), jnp.int32)
            fwd = jnp.zeros((16,), jnp.int32)
            for s in range(1, 16):
                back = plsc.load_gather(bins_v, [rot_b[s - 1]])
                dup = dup + jnp.where((lane >= s) & (back == bins), 1, 0)
                fw = plsc.load_gather(bins_v, [rot_f[s - 1]])
                fwd = fwd + jnp.where((lane <= 15 - s) & (fw == bins), 1, 0)
            return dup, fwd

        def load_bins(i):
            vr = loc_r[pl.ds(i * 16, 16)]
            valid = (i * 16 + lane) < cnt
            return vr, jnp.where(valid, lax.shift_right_logical(vr, 9),
                                 dump_bin), valid

        for v in range((_NBINS + 16) // 16):
            hist[pl.ds(v * 16, 16)] = jnp.zeros((16,), jnp.int32)

        def hist_vec(i, carry):
            _, bins, _ = load_bins(i)
            dup, fwd = dup_rank(bins)
            first = dup == 0
            tb = jnp.where(first, bins, dump_bin)
            plsc.addupdate_scatter(hist, [tb],
                                   jnp.where(first, 1 + fwd, 0))
            return carry

        lax.fori_loop(0, nvec, hist_vec, 0)

        run = jnp.int32(0)
        for v in range(_NBINS // 16):
            hv = hist[pl.ds(v * 16, 16)]
            csv = plsc.cumsum(hv)
            excl = csv - hv + run
            bounds[pl.ds(v * 16, 16)] = excl
            cursor[pl.ds(v * 16, 16)] = excl
            run = run + csv[15]

        def place_vec(i, carry):
            vr, bins, valid = load_bins(i)
            vb = loc_b[pl.ds(i * 16, 16)]
            dup, fwd = dup_rank(bins)
            first = dup == 0
            base = plsc.load_gather(cursor, [bins])
            pos = jnp.where(valid, base + dup, pad_slot)
            plsc.store_scatter(srt_r, [pos], vr)
            plsc.store_scatter(srt_b, [pos], vb)
            tb = jnp.where(first, bins, dump_bin)
            plsc.addupdate_scatter(cursor, [tb],
                                   jnp.where(first, 1 + fwd, 0))
            return carry

        lax.fori_loop(0, nvec, place_vec, 0)

        def process(k, dma_r0, slab):
            s = bounds[pl.ds(k, 16)][0]
            e = bounds[pl.ds(k + 1, 16)][0]

            def flush(sb, carry):
                base = s + sb * _FLUSH
                nit = jnp.minimum(_FLUSH, e - base)

                def fill(i, carry):
                    rr = srt_r[pl.ds(base + i, 16)][0] - dma_r0
                    for g in range(chalf // 16):
                        rows_p[i, pl.ds(g * 16, 16)] = plsc.load_gather(
                            slab, [lane + g * 16,
                                   jnp.full((16,), rr, jnp.int32)])
                    return carry

                lax.fori_loop(0, nit, fill, 0)
                for g in range(_FLUSH // 16):
                    li = base + g * 16 + lane
                    vb = srt_b[pl.ds(base + g * 16, 16)]
                    bpad[pl.ds(g * 16, 16)] = jnp.where(
                        li < e, vb + hbase, dump)
                pltpu.async_copy(rows_p, out_hbm.at[bpad], sem_s).wait()
                return carry

            lax.fori_loop(0, (e - s + _FLUSH - 1) // _FLUSH, flush, 0)

        def do_block(k, slab, sem, fire_next):
            drain(slab, sem)
            fire_next()
            process(k, pl.multiple_of(k * _BLK, 128), slab)

        def pair_loop(p, carry):
            k = p * 2
            do_block(k, slab_a, sem_a,
                     lambda: fire(pl.multiple_of((k + 1) * _BLK, 128),
                                  slab_b, sem_b))
            do_block(k + 1, slab_b, sem_b,
                     lambda: fire(pl.multiple_of((k + 2) * _BLK, 128),
                                  slab_a, sem_a))
            return carry

        lax.fori_loop(0, _R_MAIN // _BLK // 2, pair_loop, 0)
        drain(slab_a, sem_a)

        @pl.when(is_last)
        def _tail():
            process(122, 122 * _BLK, slab_a)
            s = bounds[pl.ds(tail_bin, 16)][0]
            e = bounds[pl.ds(tail_bin + 1, 16)][0]

            def tail_item(i, carry):
                rr = srt_r[pl.ds(i, 16)][0]
                bb = srt_b[pl.ds(i, 16)][0]
                pltpu.sync_copy(
                    tab_hbm.at[pl.ds(tile_r0 + rr, 1), pl.ds(c0, chalf)],
                    rows_p.at[pl.ds(0, 1), pl.ds(0, chalf)])
                pltpu.sync_copy(
                    rows_p.at[pl.ds(0, 1)],
                    out_hbm.at[pl.ds(hbase + bb, 1)])
                return carry

            lax.fori_loop(s, e, tail_item, 0)

    return sweep_kernel


@jax.jit
def kernel(user_inputs, table):
    num_rows, dim = table.shape
    batch = user_inputs.shape[0]
    sweep = _make_sweep(num_rows, dim, batch)
    y2 = sweep(user_inputs.astype(jnp.int32), table.T, table)
    return jnp.concatenate(
        [y2[:batch, : dim // 2], y2[_HPAD:_HPAD + batch, : dim // 2]],
        axis=1)


# speedup vs baseline: 1.6603x; 1.3174x over previous
import functools

import jax
import jax.numpy as jnp
from jax import lax
from jax.experimental import pallas as pl
from jax.experimental.pallas import tpu as pltpu
from jax.experimental.pallas import tpu_sc as plsc

EMBED_DIM = 64
BATCH = 16384

_info = plsc.get_sparse_core_info()
_NC, _NS = _info.num_cores, _info.num_subcores

_BLK = 1024
_R_MAIN = 62464
_FLUSH = 64
_HPAD = BATCH + 64
_NBINS = 64
_BSHIFT = 14


def _make_sweep(num_rows, dim, batch):
    chalf = dim // _NC
    mesh = plsc.VectorSubcoreMesh(core_axis_name="c", subcore_axis_name="s")

    @functools.partial(
        pl.kernel,
        mesh=mesh,
        out_type=jax.ShapeDtypeStruct((_NC * _HPAD, 128), jnp.float32),
        scratch_types=[
            pltpu.VMEM((chalf, _BLK), jnp.float32),
            pltpu.VMEM((chalf, _BLK), jnp.float32),
            pltpu.VMEM((2048,), jnp.int32),
            pltpu.VMEM((batch + 16,), jnp.int32),
            pltpu.VMEM((batch + 16,), jnp.int32),
            pltpu.VMEM((_NBINS + 16,), jnp.int32),
            pltpu.VMEM((_NBINS + 16,), jnp.int32),
            pltpu.VMEM((_NBINS + 16,), jnp.int32),
            pltpu.VMEM((16,), jnp.int32),
            pltpu.VMEM((_FLUSH, 128), jnp.float32),
            pltpu.VMEM((_FLUSH,), jnp.int32),
            pltpu.SemaphoreType.DMA,
            pltpu.SemaphoreType.DMA,
            pltpu.SemaphoreType.DMA,
        ],
        compiler_params=pltpu.CompilerParams(needs_layout_passes=False),
    )
    def sweep_kernel(idx_hbm, tt_hbm, tab_hbm, out_hbm, slab_a, slab_b,
                     idx_c, loc_k, srt_k, hist, bounds, cursor, bins_v,
                     rows_p, bpad, sem_a, sem_b, sem_s):
        core = lax.axis_index("c")
        sub = lax.axis_index("s")
        c0 = core * chalf
        tile_r0 = pl.multiple_of(sub * _R_MAIN, 128)
        is_last = sub == _NS - 1
        r_range = jnp.where(is_last, num_rows - (_NS - 1) * _R_MAIN,
                            _R_MAIN)
        hbase = core * _HPAD
        dump = hbase + batch + sub
        lane = lax.iota(jnp.int32, 16)
        pad_slot = jnp.int32(batch + 8)
        dump_bin = jnp.int32(_NBINS - 1)
        tail_bin = 62
        tail_key = jnp.int32(62976 << _BSHIFT)
        bmask = jnp.int32(batch - 1)

        def fire(dma_r0, slab, sem):
            pltpu.async_copy(
                tt_hbm.at[pl.ds(c0, chalf),
                          pl.ds(pl.multiple_of(tile_r0 + dma_r0, 128),
                                _BLK)],
                slab, sem)

        def drain(slab, sem):
            pltpu.make_async_copy(
                tt_hbm.at[pl.ds(c0, chalf), pl.ds(tile_r0, _BLK)],
                slab, sem).wait()

        fire(0, slab_a, sem_a)

        def scan_chunk(ch, cnt):
            pltpu.sync_copy(idx_hbm.at[pl.ds(ch * 2048, 2048)], idx_c)

            def scan_vec(i, cnt):
                v = idx_c[pl.ds(i * 16, 16)]
                vr = v - tile_r0
                m = (vr >= 0) & (vr < r_range)
                pop = plsc.all_reduce_population_count(m)[0]

                @pl.when(pop > 0)
                def _store():
                    b = ch * 2048 + i * 16 + lane
                    key = lax.shift_left(vr, _BSHIFT) | b
                    cs = plsc.cumsum(m.astype(jnp.int32))
                    pos = jnp.where(m, cnt + cs - 1, pad_slot)
                    plsc.store_scatter(loc_k, [pos], key)

                return cnt + pop

            return lax.fori_loop(0, 2048 // 16, scan_vec, cnt)

        cnt = lax.fori_loop(0, batch // 2048, scan_chunk, jnp.int32(0))
        nvec = (cnt + 15) // 16

        rot_b = [lax.rem(lane - s + 16, 16) for s in range(1, 16)]
        rot_f = [lax.rem(lane + s, 16) for s in range(1, 16)]

        def dup_rank(bins):
            bins_v[pl.ds(0, 16)] = bins
            dup = jnp.zeros((16,), jnp.int32)
            fwd = jnp.zeros((16,), jnp.int32)
            for s in range(1, 16):
                back = plsc.load_gather(bins_v, [rot_b[s - 1]])
                dup = dup + jnp.where((lane >= s) & (back == bins), 1, 0)
                fw = plsc.load_gather(bins_v, [rot_f[s - 1]])
                fwd = fwd + jnp.where((lane <= 15 - s) & (fw == bins), 1, 0)
            return dup, fwd

        def load_bins(i):
            key = loc_k[pl.ds(i * 16, 16)]
            valid = (i * 16 + lane) < cnt
            bins = lax.shift_right_logical(key, _BSHIFT + 10)
            bins = jnp.where(key >= tail_key, tail_bin, bins)
            return key, jnp.where(valid, bins, dump_bin), valid

        for v in range((_NBINS + 16) // 16):
            hist[pl.ds(v * 16, 16)] = jnp.zeros((16,), jnp.int32)

        def hist_vec(i, carry):
            _, bins, _ = load_bins(i)
            dup, fwd = dup_rank(bins)
            first = dup == 0
            tb = jnp.where(first, bins, dump_bin)
            plsc.addupdate_scatter(hist, [tb],
                                   jnp.where(first, 1 + fwd, 0))
            return carry

        lax.fori_loop(0, nvec, hist_vec, 0)

        run = jnp.int32(0)
        for v in range(_NBINS // 16):
            hv = hist[pl.ds(v * 16, 16)]
            csv = plsc.cumsum(hv)
            excl = csv - hv + run
            bounds[pl.ds(v * 16, 16)] = excl
            cursor[pl.ds(v * 16, 16)] = excl
            run = run + csv[15]

        def place_vec(i, carry):
            key, bins, valid = load_bins(i)
            dup, fwd = dup_rank(bins)
            first = dup == 0
            base = plsc.load_gather(cursor, [bins])
            pos = jnp.where(valid, base + dup, pad_slot)
            plsc.store_scatter(srt_k, [pos], key)
            tb = jnp.where(first, bins, dump_bin)
            plsc.addupdate_scatter(cursor, [tb],
                                   jnp.where(first, 1 + fwd, 0))
            return carry

        lax.fori_loop(0, nvec, place_vec, 0)

        def process(k, dma_r0, slab):
            s = bounds[pl.ds(k, 16)][0]
            e = bounds[pl.ds(k + 1, 16)][0]

            def flush(sb, carry):
                base = s + sb * _FLUSH
                for g in range(_FLUSH // 16):
                    kv = srt_k[pl.ds(base + g * 16, 16)]
                    li = base + g * 16 + lane
                    ok = li < e
                    rr = lax.shift_right_logical(kv, _BSHIFT) - dma_r0
                    rr = jnp.where(ok, rr, 0)
                    row16 = lane + g * 16
                    for c in range(chalf):
                        cc = jnp.full((16,), c, jnp.int32)
                        vals = plsc.load_gather(slab, [cc, rr])
                        plsc.store_scatter(rows_p, [row16, cc], vals)
                    bpad[pl.ds(g * 16, 16)] = jnp.where(
                        ok, (kv & bmask) + hbase, dump)
                pltpu.async_copy(rows_p, out_hbm.at[bpad], sem_s).wait()
                return carry

            lax.fori_loop(0, (e - s + _FLUSH - 1) // _FLUSH, flush, 0)

        def do_block(k, slab, sem, fire_next):
            drain(slab, sem)
            fire_next()
            process(k, pl.multiple_of(k * _BLK, 128), slab)

        def pair_loop(p, carry):
            k = p * 2
            do_block(k, slab_a, sem_a,
                     lambda: fire(pl.multiple_of((k + 1) * _BLK, 128),
                                  slab_b, sem_b))
            do_block(k + 1, slab_b, sem_b,
                     lambda: fire(pl.multiple_of((k + 2) * _BLK, 128),
                                  slab_a, sem_a))
            return carry

        lax.fori_loop(0, 30, pair_loop, 0)
        drain(slab_a, sem_a)
        process(60, 60 * _BLK, slab_a)

        @pl.when(is_last)
        def _tail():
            fire(61952, slab_b, sem_b)
            drain(slab_b, sem_b)
            process(61, 61952, slab_b)
            s = bounds[pl.ds(tail_bin, 16)][0]
            e = bounds[pl.ds(tail_bin + 1, 16)][0]

            def tail_item(i, carry):
                kv = srt_k[pl.ds(i, 16)][0]
                rr = lax.shift_right_logical(kv, _BSHIFT)
                bb = kv & bmask
                pltpu.sync_copy(
                    tab_hbm.at[pl.ds(tile_r0 + rr, 1), pl.ds(c0, chalf)],
                    rows_p.at[pl.ds(0, 1), pl.ds(0, chalf)])
                pltpu.sync_copy(
                    rows_p.at[pl.ds(0, 1)],
                    out_hbm.at[pl.ds(hbase + bb, 1)])
                return carry

            lax.fori_loop(s, e, tail_item, 0)

    return sweep_kernel


@jax.jit
def kernel(user_inputs, table):
    num_rows, dim = table.shape
    batch = user_inputs.shape[0]
    sweep = _make_sweep(num_rows, dim, batch)
    y2 = sweep(user_inputs.astype(jnp.int32), table.T, table)
    return jnp.concatenate(
        [y2[:batch, : dim // 2], y2[_HPAD:_HPAD + batch, : dim // 2]],
        axis=1)


# speedup vs baseline: 3.3917x; 2.0429x over previous
import functools

import jax
import jax.numpy as jnp
from jax import lax
from jax.experimental import pallas as pl
from jax.experimental.pallas import tpu as pltpu
from jax.experimental.pallas import tpu_sc as plsc

EMBED_DIM = 64
BATCH = 16384

_info = plsc.get_sparse_core_info()
_NC, _NS = _info.num_cores, _info.num_subcores
_NW = _NC * _NS


def _make_gather(dim, batch):
    b_per_w = batch // _NW
    mesh = plsc.VectorSubcoreMesh(core_axis_name="c", subcore_axis_name="s")

    @functools.partial(
        pl.kernel,
        mesh=mesh,
        out_type=jax.ShapeDtypeStruct((batch, dim), jnp.float32),
        scratch_types=[
            pltpu.VMEM((b_per_w,), jnp.int32),
            pltpu.VMEM((b_per_w, dim), jnp.float32),
            pltpu.SemaphoreType.DMA,
            pltpu.SemaphoreType.DMA,
            pltpu.SemaphoreType.DMA,
            pltpu.SemaphoreType.DMA,
        ],
    )
    def gather_kernel(idx_hbm, table_hbm, out_hbm, idx_v, rows_v,
                      sem0, sem1, sem2, sem3):
        sems = [sem0, sem1, sem2, sem3]
        wid = lax.axis_index("s") * _NC + lax.axis_index("c")
        base = wid * b_per_w
        pltpu.sync_copy(idx_hbm.at[pl.ds(base, b_per_w)], idx_v)

        def fire(g, carry):
            vec = idx_v[pl.ds(g * 16, 16)]
            for l in range(16):
                r = vec[l]
                pltpu.async_copy(
                    table_hbm.at[pl.ds(r, 1)],
                    rows_v.at[pl.ds(g * 16 + l, 1)],
                    sems[l % 4],
                )
            return carry

        lax.fori_loop(0, b_per_w // 16, fire, 0)
        for q in range(4):
            pltpu.make_async_copy(
                table_hbm.at[pl.ds(0, b_per_w // 4)],
                rows_v.at[pl.ds(0, b_per_w // 4)], sems[q]
            ).wait()
        pltpu.sync_copy(rows_v, out_hbm.at[pl.ds(base, b_per_w)])

    return gather_kernel


@jax.jit
def kernel(user_inputs, table):
    gather = _make_gather(table.shape[1], user_inputs.shape[0])
    return gather(user_inputs.astype(jnp.int32), table)


# speedup vs baseline: 3.3978x; 1.0018x over previous
import functools

import jax
import jax.numpy as jnp
from jax import lax
from jax.experimental import pallas as pl
from jax.experimental.pallas import tpu as pltpu
from jax.experimental.pallas import tpu_sc as plsc

EMBED_DIM = 64
BATCH = 16384

_info = plsc.get_sparse_core_info()
_NC, _NS = _info.num_cores, _info.num_subcores
_NW = _NC * _NS


def _make_gather(dim, batch):
    b_per_w = batch // _NW
    mesh = plsc.VectorSubcoreMesh(core_axis_name="c", subcore_axis_name="s")

    @functools.partial(
        pl.kernel,
        mesh=mesh,
        out_type=jax.ShapeDtypeStruct((batch, dim), jnp.float32),
        scratch_types=[
            pltpu.VMEM((b_per_w,), jnp.int32),
            pltpu.VMEM((b_per_w, dim), jnp.float32),
            pltpu.SemaphoreType.DMA,
        ],
    )
    def gather_kernel(idx_hbm, table_hbm, out_hbm, idx_v, rows_v, sem):
        wid = lax.axis_index("s") * _NC + lax.axis_index("c")
        base = wid * b_per_w
        pltpu.sync_copy(idx_hbm.at[pl.ds(base, b_per_w)], idx_v)

        def fire(g, carry):
            vec = idx_v[pl.ds(g * 16, 16)]
            for l in range(16):
                r = vec[l]
                pltpu.async_copy(
                    table_hbm.at[pl.ds(r, 1)],
                    rows_v.at[pl.ds(g * 16 + l, 1)],
                    sem,
                )
            return carry

        lax.fori_loop(0, b_per_w // 16, fire, 0)
        pltpu.make_async_copy(
            table_hbm.at[pl.ds(0, b_per_w)], rows_v, sem
        ).wait()
        pltpu.sync_copy(rows_v, out_hbm.at[pl.ds(base, b_per_w)])

    return gather_kernel


@jax.jit
def kernel(user_inputs, table):
    gather = _make_gather(table.shape[1], user_inputs.shape[0])
    return gather(user_inputs.astype(jnp.int32), table)
